# baseline (speedup 1.0000x reference)
import functools

import jax
import jax.numpy as jnp
from jax import lax
from jax.experimental import pallas as pl
from jax.experimental.pallas import tpu as pltpu
from jax.experimental.pallas import tpu_sc as plsc

B, C, H, W = 2, 96, 384, 384
QK_DIM = 96
NUM_HEADS = 3
K_SP = 576
TOPK = 64
HEAD_DIM = QK_DIM // NUM_HEADS
SCALE = HEAD_DIM ** (-0.5)
HW = H * W
NSP = B * K_SP


NTOK = B * K_SP * TOPK
NW = 32
TPW = NTOK // NW
GCH = 18
assert GCH * 128 == TPW


CP = 128
NBUF = 4


def _gather_body(xt_hbm, idxg_hbm, out_hbm, idx_v, *bufsems):
    cid = lax.axis_index("c")
    sid = lax.axis_index("s")
    wid = sid * 2 + cid
    pltpu.sync_copy(idxg_hbm.at[wid], idx_v)
    bufs = bufsems[:NBUF]
    gsems = bufsems[NBUF:2 * NBUF]
    wsems = bufsems[2 * NBUF:]

    def out_slice(j):
        return out_hbm.at[pl.ds(wid * TPW + j * 128, 128)]

    for j in range(NBUF):
        pltpu.async_copy(xt_hbm.at[idx_v.at[j]], bufs[j], gsems[j])
    for j in range(GCH):
        b = j % NBUF
        pltpu.make_async_copy(xt_hbm.at[idx_v.at[j]], bufs[b], gsems[b]).wait()
        pltpu.async_copy(bufs[b], out_slice(j), wsems[b])
        if j + NBUF < GCH:
            pltpu.make_async_copy(bufs[b], out_slice(j), wsems[b]).wait()
            pltpu.async_copy(xt_hbm.at[idx_v.at[j + NBUF]], bufs[b], gsems[b])
    for j in range(GCH - NBUF, GCH):
        b = j % NBUF
        pltpu.make_async_copy(bufs[b], out_slice(j), wsems[b]).wait()


@functools.cache
def _gather_sc():
    return pl.kernel(
        _gather_body,
        mesh=plsc.VectorSubcoreMesh(core_axis_name="c", subcore_axis_name="s"),
        out_type=jax.ShapeDtypeStruct((NTOK, CP), jnp.float32),
        scratch_types=(
            [pltpu.VMEM((GCH, 128), jnp.int32)]
            + [pltpu.VMEM((128, CP), jnp.float32) for _ in range(NBUF)]
            + [pltpu.SemaphoreType.DMA for _ in range(2 * NBUF)]
        ),
    )


TPB = K_SP * TOPK
TPT = TPB // 16
SCH = TPT // 128
GW = 8
NGP = C // GW
EPT = TPT * GW
ECH = EPT // 128
SPW = HW * GW
EPS = SPW // 16


def _scatter_body(tok_hbm, idxe_hbm, idxc_hbm, consts_hbm, acc_hbm, cnt_hbm,
                  acc_sp, idxe_v, idxc_v, tok_v, zero_v):
    cid = lax.axis_index("c")
    sid = lax.axis_index("s")
    pltpu.sync_copy(idxe_hbm.at[cid, sid], idxe_v)
    pltpu.sync_copy(idxc_hbm.at[cid, sid], idxc_v)
    pltpu.sync_copy(consts_hbm.at[0], zero_v)
    pltpu.sync_copy(consts_hbm.at[1], tok_v.at[pl.ds(0, TPT)])

    for k in range(HW // 16 // TPT):
        pltpu.sync_copy(zero_v,
                        acc_sp.at[pl.ds(sid * (HW // 16) + k * TPT, TPT)])
    plsc.subcore_barrier()
    for j in range(SCH):
        pltpu.sync_copy(tok_v.at[pl.ds(j * 128, 128)],
                        acc_sp.at[idxc_v.at[j]], add=True)
    plsc.subcore_barrier()
    pltpu.sync_copy(acc_sp.at[pl.ds(sid * (HW // 16), HW // 16)],
                    cnt_hbm.at[cid, sid])
    plsc.subcore_barrier()

    for g in range(NGP):
        for k in range(EPS // TPT):
            pltpu.sync_copy(zero_v,
                            acc_sp.at[pl.ds(sid * EPS + k * TPT, TPT)])
        plsc.subcore_barrier()
        pltpu.sync_copy(tok_hbm.at[cid, g, pl.ds(sid * EPT, EPT)], tok_v)

        def chunk(j, _):
            pltpu.sync_copy(tok_v.at[pl.ds(j * 128, 128)],
                            acc_sp.at[idxe_v.at[j]], add=True)
            return 0

        lax.fori_loop(0, ECH, chunk, 0)
        plsc.subcore_barrier()
        pltpu.sync_copy(acc_sp.at[pl.ds(sid * EPS, EPS)],
                        acc_hbm.at[cid, g, sid])
        plsc.subcore_barrier()


@functools.cache
def _scatter_sc():
    return pl.kernel(
        _scatter_body,
        mesh=plsc.VectorSubcoreMesh(core_axis_name="c", subcore_axis_name="s"),
        out_type=[
            jax.ShapeDtypeStruct((B, NGP, 16, EPS), jnp.float32),
            jax.ShapeDtypeStruct((B, 16, HW // 16), jnp.float32),
        ],
        scratch_types=[
            pltpu.VMEM_SHARED((SPW,), jnp.float32),
            pltpu.VMEM((ECH, 128), jnp.int32),
            pltpu.VMEM((SCH, 128), jnp.int32),
            pltpu.VMEM((EPT,), jnp.float32),
            pltpu.VMEM((TPT,), jnp.float32),
        ],
    )


PB = 512
NPIX = B * HW


PPB = HW // PB


def _lnv_body(x_ref, vw_ref, lnw_ref, lnb_ref, xtp_ref, v_ref):
    xc = x_ref[0]
    xcp = jnp.concatenate(
        [xc, jnp.zeros((CP - C, PB), jnp.float32)], axis=0)
    xtpb = jnp.transpose(xcp, (1, 0))
    xt = xtpb[:, :C]
    u = jnp.mean(xt, axis=1, keepdims=True)
    var = jnp.mean((xt - u) ** 2, axis=1, keepdims=True)
    xn = (xt - u) * jax.lax.rsqrt(var + 1e-6)
    xn = xn * lnw_ref[...] + lnb_ref[...]
    dn = (((1,), (1,)), ((), ()))
    v = jax.lax.dot_general(xn, vw_ref[...], dn, preferred_element_type=jnp.float32)
    xtp_ref[...] = xtpb
    v_ref[...] = v


def _lnv(x, v_w, ln_w, ln_b):
    return pl.pallas_call(
        _lnv_body,
        grid=(NPIX // PB,),
        in_specs=[
            pl.BlockSpec((1, C, PB), lambda i: (i // PPB, 0, i % PPB)),
            pl.BlockSpec((C, C), lambda i: (0, 0)),
            pl.BlockSpec((1, C), lambda i: (0, 0)),
            pl.BlockSpec((1, C), lambda i: (0, 0)),
        ],
        out_specs=[
            pl.BlockSpec((PB, CP), lambda i: (i, 0)),
            pl.BlockSpec((PB, C), lambda i: (i, 0)),
        ],
        out_shape=[
            jax.ShapeDtypeStruct((NPIX, CP), jnp.float32),
            jax.ShapeDtypeStruct((NPIX, C), jnp.float32),
        ],
    )(x.reshape(B, C, HW), v_w, ln_w.reshape(1, C), ln_b.reshape(1, C))


def _merge_body(acc_ref, cnt_ref, v_ref, out_ref):
    cnt = cnt_ref[...]
    acc = acc_ref[...]
    mean = acc / jnp.maximum(cnt, 1.0)
    merged = jnp.where(cnt > 1e-5, mean, v_ref[...])
    mp = jnp.concatenate(
        [merged, jnp.zeros((PB, CP - C), jnp.float32)], axis=1)
    mt = jnp.transpose(mp, (1, 0))
    out_ref[0] = mt[:C, :]


def _merge(acc, cnt, v_full):
    return pl.pallas_call(
        _merge_body,
        grid=(NPIX // PB,),
        in_specs=[
            pl.BlockSpec((PB, C), lambda i: (i, 0)),
            pl.BlockSpec((PB, 1), lambda i: (i, 0)),
            pl.BlockSpec((PB, C), lambda i: (i, 0)),
        ],
        out_specs=pl.BlockSpec((1, C, PB), lambda i: (i // PPB, 0, i % PPB)),
        out_shape=jax.ShapeDtypeStruct((B, C, HW), jnp.float32),
    )(acc, cnt.reshape(NPIX, 1), v_full)


SPB = 8
NT = SPB * TOPK


def _attn_body(xg_ref, sims_ref, qw_ref, kw_ref, vw_ref, lnw_ref, lnb_ref, out_ref):
    xg = xg_ref[:, :C]
    u = jnp.mean(xg, axis=1, keepdims=True)
    var = jnp.mean((xg - u) ** 2, axis=1, keepdims=True)
    xn = (xg - u) * jax.lax.rsqrt(var + 1e-6)
    xn = xn * lnw_ref[...] + lnb_ref[...]
    dn = (((1,), (1,)), ((), ()))
    q = jax.lax.dot_general(xn, qw_ref[...], dn, preferred_element_type=jnp.float32)
    k = jax.lax.dot_general(xn, kw_ref[...], dn, preferred_element_type=jnp.float32)
    v = jax.lax.dot_general(xn, vw_ref[...], dn, preferred_element_type=jnp.float32)
    s_col = sims_ref[...].reshape(NT, 1)
    vw_all = s_col * v
    ones_row = jnp.ones((1, HEAD_DIM), dtype=jnp.float32)
    for p in range(SPB):
        rows = slice(p * TOPK, (p + 1) * TOPK)
        sp_col = s_col[rows]
        for h in range(NUM_HEADS):
            cols = slice(h * HEAD_DIM, (h + 1) * HEAD_DIM)
            qh = q[rows, cols]
            kh = k[rows, cols]
            qq = jnp.sum(qh * qh, axis=1, keepdims=True)
            kk = jax.lax.dot_general(ones_row, kh * kh, dn,
                                     preferred_element_type=jnp.float32)
            qk = jax.lax.dot_general(qh, kh, dn,
                                     preferred_element_type=jnp.float32)
            d2 = qq + kk - 2.0 * qk
            dist = jnp.sqrt(jnp.maximum(d2, 1e-12))
            a = -SCALE * dist
            m = jnp.max(a, axis=1, keepdims=True)
            e = jnp.exp(a - m)
            pr = e / jnp.sum(e, axis=1, keepdims=True)
            oh = jax.lax.dot_general(pr, vw_all[rows, cols],
                                     (((1,), (0,)), ((), ())),
                                     preferred_element_type=jnp.float32)
            out_ref[rows, cols] = sp_col * oh


def _attention(xg2, simsT, q_w, k_w, v_w, ln_w, ln_b):
    return pl.pallas_call(
        _attn_body,
        grid=(NSP // SPB,),
        in_specs=[
            pl.BlockSpec((NT, CP), lambda i: (i, 0)),
            pl.BlockSpec((SPB, TOPK, 1), lambda i: (i, 0, 0)),
            pl.BlockSpec((QK_DIM, C), lambda i: (0, 0)),
            pl.BlockSpec((QK_DIM, C), lambda i: (0, 0)),
            pl.BlockSpec((C, C), lambda i: (0, 0)),
            pl.BlockSpec((1, C), lambda i: (0, 0)),
            pl.BlockSpec((1, C), lambda i: (0, 0)),
        ],
        out_specs=pl.BlockSpec((NT, C), lambda i: (i, 0)),
        out_shape=jax.ShapeDtypeStruct((NSP * TOPK, C), jnp.float32),
    )(xg2, simsT, q_w, k_w, v_w, ln_w, ln_b)


def kernel(x, sims, mask, ln_w, ln_b, q_w, k_w, v_w, indices, labels, num_spixels):
    idx = indices.reshape(B, K_SP * TOPK)
    idx_g = (idx + jnp.arange(B, dtype=jnp.int32)[:, None] * HW)
    xtp, v_full = _lnv(x, v_w, ln_w, ln_b)
    xg = _gather_sc()(xtp, idx_g.reshape(NW, GCH, 128))

    out_tok = _attention(
        xg,
        sims.reshape(NSP, TOPK, 1),
        q_w, k_w, v_w, ln_w.reshape(1, C), ln_b.reshape(1, C))
    out_cg = (out_tok.reshape(B, TPB, NGP, GW)
              .transpose(0, 2, 1, 3).reshape(B, NGP, TPB * GW))
    idx_e = (idx[..., None] * GW
             + jnp.arange(GW, dtype=jnp.int32)).reshape(B, 16, ECH, 128)
    idx_c = idx.reshape(B, 16, SCH, 128)
    consts = jnp.stack([jnp.zeros((TPT,), jnp.float32),
                        jnp.ones((TPT,), jnp.float32)])
    acc_f, cnt_f = _scatter_sc()(out_cg, idx_e, idx_c, consts)
    acc = (acc_f.reshape(B, NGP, HW, GW)
           .transpose(0, 2, 1, 3).reshape(NPIX, C))
    merged = _merge(acc, cnt_f.reshape(NPIX), v_full)
    return merged.reshape(B, C, H, W)


# speedup vs baseline: 1.0971x; 1.0971x over previous
import functools

import jax
import jax.numpy as jnp
from jax import lax
from jax.experimental import pallas as pl
from jax.experimental.pallas import tpu as pltpu
from jax.experimental.pallas import tpu_sc as plsc

B, C, H, W = 2, 96, 384, 384
QK_DIM = 96
NUM_HEADS = 3
K_SP = 576
TOPK = 64
HEAD_DIM = QK_DIM // NUM_HEADS
SCALE = HEAD_DIM ** (-0.5)
HW = H * W
NSP = B * K_SP


NTOK = B * K_SP * TOPK
NW = 32
TPW = NTOK // NW
GCH = 18
assert GCH * 128 == TPW


CP = 128
NBUF = 4


def _gather_body(xt_hbm, idxg_hbm, out_hbm, idx_v, *bufsems):
    cid = lax.axis_index("c")
    sid = lax.axis_index("s")
    wid = sid * 2 + cid
    pltpu.sync_copy(idxg_hbm.at[wid], idx_v)
    bufs = bufsems[:NBUF]
    gsems = bufsems[NBUF:2 * NBUF]
    wsems = bufsems[2 * NBUF:]

    def out_slice(j):
        return out_hbm.at[pl.ds(wid * TPW + j * 128, 128)]

    for j in range(NBUF):
        pltpu.async_copy(xt_hbm.at[idx_v.at[j]], bufs[j], gsems[j])
    for j in range(GCH):
        b = j % NBUF
        pltpu.make_async_copy(xt_hbm.at[idx_v.at[j]], bufs[b], gsems[b]).wait()
        pltpu.async_copy(bufs[b], out_slice(j), wsems[b])
        if j + NBUF < GCH:
            pltpu.make_async_copy(bufs[b], out_slice(j), wsems[b]).wait()
            pltpu.async_copy(xt_hbm.at[idx_v.at[j + NBUF]], bufs[b], gsems[b])
    for j in range(GCH - NBUF, GCH):
        b = j % NBUF
        pltpu.make_async_copy(bufs[b], out_slice(j), wsems[b]).wait()


@functools.cache
def _gather_sc():
    return pl.kernel(
        _gather_body,
        mesh=plsc.VectorSubcoreMesh(core_axis_name="c", subcore_axis_name="s"),
        out_type=jax.ShapeDtypeStruct((NTOK, CP), jnp.float32),
        scratch_types=(
            [pltpu.VMEM((GCH, 128), jnp.int32)]
            + [pltpu.VMEM((128, CP), jnp.float32) for _ in range(NBUF)]
            + [pltpu.SemaphoreType.DMA for _ in range(2 * NBUF)]
        ),
    )


TPB = K_SP * TOPK
TPT = TPB // 16
SCH = TPT // 128
GW = 8
NGP = C // GW
EPT = TPT * GW
ECH = EPT // 128
SPW = HW * GW
EPS = SPW // 16


def _scatter_body(tok_hbm, idxe_hbm, idxc_hbm, consts_hbm, acc_hbm, cnt_hbm,
                  acc_sp, idxe_v, idxc_v, tok_v, zero_v):
    cid = lax.axis_index("c")
    sid = lax.axis_index("s")
    pltpu.sync_copy(idxe_hbm.at[cid, sid], idxe_v)
    pltpu.sync_copy(idxc_hbm.at[cid, sid], idxc_v)
    pltpu.sync_copy(consts_hbm.at[0], zero_v)
    pltpu.sync_copy(consts_hbm.at[1], tok_v.at[pl.ds(0, TPT)])

    for k in range(HW // 16 // TPT):
        pltpu.sync_copy(zero_v,
                        acc_sp.at[pl.ds(sid * (HW // 16) + k * TPT, TPT)])
    plsc.subcore_barrier()
    for j in range(SCH):
        pltpu.sync_copy(tok_v.at[pl.ds(j * 128, 128)],
                        acc_sp.at[idxc_v.at[j]], add=True)
    plsc.subcore_barrier()
    pltpu.sync_copy(acc_sp.at[pl.ds(sid * (HW // 16), HW // 16)],
                    cnt_hbm.at[cid, sid])
    plsc.subcore_barrier()

    for g in range(NGP):
        for k in range(EPS // TPT):
            pltpu.sync_copy(zero_v,
                            acc_sp.at[pl.ds(sid * EPS + k * TPT, TPT)])
        plsc.subcore_barrier()
        pltpu.sync_copy(tok_hbm.at[cid, g, pl.ds(sid * EPT, EPT)], tok_v)

        def chunk(j, _):
            pltpu.sync_copy(tok_v.at[pl.ds(j * 128, 128)],
                            acc_sp.at[idxe_v.at[j]], add=True)
            return 0

        lax.fori_loop(0, ECH, chunk, 0)
        plsc.subcore_barrier()
        pltpu.sync_copy(acc_sp.at[pl.ds(sid * EPS, EPS)],
                        acc_hbm.at[cid, g, sid])
        plsc.subcore_barrier()


@functools.cache
def _scatter_sc():
    return pl.kernel(
        _scatter_body,
        mesh=plsc.VectorSubcoreMesh(core_axis_name="c", subcore_axis_name="s"),
        out_type=[
            jax.ShapeDtypeStruct((B, NGP, 16, EPS), jnp.float32),
            jax.ShapeDtypeStruct((B, 16, HW // 16), jnp.float32),
        ],
        scratch_types=[
            pltpu.VMEM_SHARED((SPW,), jnp.float32),
            pltpu.VMEM((ECH, 128), jnp.int32),
            pltpu.VMEM((SCH, 128), jnp.int32),
            pltpu.VMEM((EPT,), jnp.float32),
            pltpu.VMEM((TPT,), jnp.float32),
        ],
    )


PB = 512
NPIX = B * HW


PPB = HW // PB


def _lnv_body(x_ref, vw_ref, lnw_ref, lnb_ref, xtp_ref, v_ref):
    xc = x_ref[0]
    xcp = jnp.concatenate(
        [xc, jnp.zeros((CP - C, PB), jnp.float32)], axis=0)
    xtpb = jnp.transpose(xcp, (1, 0))
    xt = xtpb[:, :C]
    u = jnp.mean(xt, axis=1, keepdims=True)
    var = jnp.mean((xt - u) ** 2, axis=1, keepdims=True)
    xn = (xt - u) * jax.lax.rsqrt(var + 1e-6)
    xn = xn * lnw_ref[...] + lnb_ref[...]
    dn = (((1,), (1,)), ((), ()))
    v = jax.lax.dot_general(xn, vw_ref[...], dn, preferred_element_type=jnp.float32)
    xtp_ref[...] = xtpb
    v_ref[...] = v


def _lnv(x, v_w, ln_w, ln_b):
    return pl.pallas_call(
        _lnv_body,
        grid=(NPIX // PB,),
        in_specs=[
            pl.BlockSpec((1, C, PB), lambda i: (i // PPB, 0, i % PPB)),
            pl.BlockSpec((C, C), lambda i: (0, 0)),
            pl.BlockSpec((1, C), lambda i: (0, 0)),
            pl.BlockSpec((1, C), lambda i: (0, 0)),
        ],
        out_specs=[
            pl.BlockSpec((PB, CP), lambda i: (i, 0)),
            pl.BlockSpec((PB, C), lambda i: (i, 0)),
        ],
        out_shape=[
            jax.ShapeDtypeStruct((NPIX, CP), jnp.float32),
            jax.ShapeDtypeStruct((NPIX, C), jnp.float32),
        ],
    )(x.reshape(B, C, HW), v_w, ln_w.reshape(1, C), ln_b.reshape(1, C))


def _merge_body(acc_ref, cnt_ref, v_ref, out_ref):
    cnt = cnt_ref[...]
    acc = acc_ref[...]
    mean = acc / jnp.maximum(cnt, 1.0)
    merged = jnp.where(cnt > 1e-5, mean, v_ref[...])
    mp = jnp.concatenate(
        [merged, jnp.zeros((PB, CP - C), jnp.float32)], axis=1)
    mt = jnp.transpose(mp, (1, 0))
    out_ref[0] = mt[:C, :]


def _merge(acc, cnt, v_full):
    return pl.pallas_call(
        _merge_body,
        grid=(NPIX // PB,),
        in_specs=[
            pl.BlockSpec((PB, C), lambda i: (i, 0)),
            pl.BlockSpec((PB, 1), lambda i: (i, 0)),
            pl.BlockSpec((PB, C), lambda i: (i, 0)),
        ],
        out_specs=pl.BlockSpec((1, C, PB), lambda i: (i // PPB, 0, i % PPB)),
        out_shape=jax.ShapeDtypeStruct((B, C, HW), jnp.float32),
    )(acc, cnt.reshape(NPIX, 1), v_full)


SPB = 8
NT = SPB * TOPK


def _attn_body(xg_ref, sims_ref, aw_ref, bqw_ref, bkw_ref, vw_ref,
               lnw_ref, lnb_ref, out_ref):
    xg = xg_ref[:, :C]
    u = jnp.mean(xg, axis=1, keepdims=True)
    var = jnp.mean((xg - u) ** 2, axis=1, keepdims=True)
    xn = (xg - u) * jax.lax.rsqrt(var + 1e-6)
    xn = xn * lnw_ref[...] + lnb_ref[...]
    nn = (((1,), (0,)), ((), ()))
    nt = (((1,), (1,)), ((), ()))
    y = jax.lax.dot_general(xn, aw_ref[...], nn, preferred_element_type=jnp.float32)
    zq = jax.lax.dot_general(xn, bqw_ref[...], nn, preferred_element_type=jnp.float32)
    zk = jax.lax.dot_general(xn, bkw_ref[...], nn, preferred_element_type=jnp.float32)
    v = jax.lax.dot_general(xn, vw_ref[...], nt, preferred_element_type=jnp.float32)
    s_col = sims_ref[...].reshape(NT, 1)
    vw_all = s_col * v
    ones_row = jnp.ones((1, C), dtype=jnp.float32)
    lane = jax.lax.broadcasted_iota(jnp.int32, (1, C), 1)
    qq_cols = []
    zkx = []
    vmask = []
    for h in range(NUM_HEADS):
        cols = slice(h * C, (h + 1) * C)
        qq_cols.append(jnp.sum(zq[:, cols] * xn, axis=1, keepdims=True))
        zkx.append(zk[:, cols] * xn)
        vmask.append(vw_all * (lane // HEAD_DIM == h).astype(jnp.float32))
    for p in range(SPB):
        rows = slice(p * TOPK, (p + 1) * TOPK)
        xr = xn[rows]
        sp_col = s_col[rows]
        parts = []
        for h in range(NUM_HEADS):
            qk = jax.lax.dot_general(y[rows, h * C:(h + 1) * C], xr, nt,
                                     preferred_element_type=jnp.float32)
            kk = jax.lax.dot_general(ones_row, zkx[h][rows], nt,
                                     preferred_element_type=jnp.float32)
            d2 = qq_cols[h][rows] + kk - 2.0 * qk
            dist = jnp.sqrt(jnp.maximum(d2, 1e-12))
            a = -SCALE * dist
            m = jnp.max(a, axis=1, keepdims=True)
            e = jnp.exp(a - m)
            parts.append(e / jnp.sum(e, axis=1, keepdims=True))
        attn_cat = jnp.concatenate(parts, axis=1)
        vbd = jnp.concatenate([vmask[h][rows] for h in range(NUM_HEADS)],
                              axis=0)
        oh = jax.lax.dot_general(attn_cat, vbd, nn,
                                 preferred_element_type=jnp.float32)
        out_ref[rows, :] = sp_col * oh


def _attention(xg2, simsT, q_w, k_w, v_w, ln_w, ln_b):
    heads_q = q_w.reshape(NUM_HEADS, HEAD_DIM, C)
    heads_k = k_w.reshape(NUM_HEADS, HEAD_DIM, C)
    a_cat = jnp.concatenate(
        [heads_q[h].T @ heads_k[h] for h in range(NUM_HEADS)], axis=1)
    bq_cat = jnp.concatenate(
        [heads_q[h].T @ heads_q[h] for h in range(NUM_HEADS)], axis=1)
    bk_cat = jnp.concatenate(
        [heads_k[h].T @ heads_k[h] for h in range(NUM_HEADS)], axis=1)
    return pl.pallas_call(
        _attn_body,
        grid=(NSP // SPB,),
        in_specs=[
            pl.BlockSpec((NT, CP), lambda i: (i, 0)),
            pl.BlockSpec((SPB, TOPK, 1), lambda i: (i, 0, 0)),
            pl.BlockSpec((C, NUM_HEADS * C), lambda i: (0, 0)),
            pl.BlockSpec((C, NUM_HEADS * C), lambda i: (0, 0)),
            pl.BlockSpec((C, NUM_HEADS * C), lambda i: (0, 0)),
            pl.BlockSpec((C, C), lambda i: (0, 0)),
            pl.BlockSpec((1, C), lambda i: (0, 0)),
            pl.BlockSpec((1, C), lambda i: (0, 0)),
        ],
        out_specs=pl.BlockSpec((NT, C), lambda i: (i, 0)),
        out_shape=jax.ShapeDtypeStruct((NSP * TOPK, C), jnp.float32),
    )(xg2, simsT, a_cat, bq_cat, bk_cat, v_w, ln_w.reshape(1, C),
      ln_b.reshape(1, C))


def kernel(x, sims, mask, ln_w, ln_b, q_w, k_w, v_w, indices, labels, num_spixels):
    idx = indices.reshape(B, K_SP * TOPK)
    idx_g = (idx + jnp.arange(B, dtype=jnp.int32)[:, None] * HW)
    xtp, v_full = _lnv(x, v_w, ln_w, ln_b)
    xg = _gather_sc()(xtp, idx_g.reshape(NW, GCH, 128))

    out_tok = _attention(
        xg,
        sims.reshape(NSP, TOPK, 1),
        q_w, k_w, v_w, ln_w.reshape(1, C), ln_b.reshape(1, C))
    out_cg = (out_tok.reshape(B, TPB, NGP, GW)
              .transpose(0, 2, 1, 3).reshape(B, NGP, TPB * GW))
    idx_e = (idx[..., None] * GW
             + jnp.arange(GW, dtype=jnp.int32)).reshape(B, 16, ECH, 128)
    idx_c = idx.reshape(B, 16, SCH, 128)
    consts = jnp.stack([jnp.zeros((TPT,), jnp.float32),
                        jnp.ones((TPT,), jnp.float32)])
    acc_f, cnt_f = _scatter_sc()(out_cg, idx_e, idx_c, consts)
    acc = (acc_f.reshape(B, NGP, HW, GW)
           .transpose(0, 2, 1, 3).reshape(NPIX, C))
    merged = _merge(acc, cnt_f.reshape(NPIX), v_full)
    return merged.reshape(B, C, H, W)


# speedup vs baseline: 1.1944x; 1.0887x over previous
import functools

import jax
import jax.numpy as jnp
from jax import lax
from jax.experimental import pallas as pl
from jax.experimental.pallas import tpu as pltpu
from jax.experimental.pallas import tpu_sc as plsc

B, C, H, W = 2, 96, 384, 384
QK_DIM = 96
NUM_HEADS = 3
K_SP = 576
TOPK = 64
HEAD_DIM = QK_DIM // NUM_HEADS
SCALE = HEAD_DIM ** (-0.5)
HW = H * W
NSP = B * K_SP


NTOK = B * K_SP * TOPK
NW = 32
TPW = NTOK // NW
GCH = 18
assert GCH * 128 == TPW


CP = 128
NBUF = 4


def _gather_body(xt_hbm, idxg_hbm, out_hbm, idx_v, *bufsems):
    cid = lax.axis_index("c")
    sid = lax.axis_index("s")
    wid = sid * 2 + cid
    pltpu.sync_copy(idxg_hbm.at[wid], idx_v)
    bufs = bufsems[:NBUF]
    gsems = bufsems[NBUF:2 * NBUF]
    wsems = bufsems[2 * NBUF:]

    def out_slice(j):
        return out_hbm.at[pl.ds(wid * TPW + j * 128, 128)]

    for j in range(NBUF):
        pltpu.async_copy(xt_hbm.at[idx_v.at[j]], bufs[j], gsems[j])
    for j in range(GCH):
        b = j % NBUF
        pltpu.make_async_copy(xt_hbm.at[idx_v.at[j]], bufs[b], gsems[b]).wait()
        pltpu.async_copy(bufs[b], out_slice(j), wsems[b])
        if j + NBUF < GCH:
            pltpu.make_async_copy(bufs[b], out_slice(j), wsems[b]).wait()
            pltpu.async_copy(xt_hbm.at[idx_v.at[j + NBUF]], bufs[b], gsems[b])
    for j in range(GCH - NBUF, GCH):
        b = j % NBUF
        pltpu.make_async_copy(bufs[b], out_slice(j), wsems[b]).wait()


@functools.cache
def _gather_sc():
    return pl.kernel(
        _gather_body,
        mesh=plsc.VectorSubcoreMesh(core_axis_name="c", subcore_axis_name="s"),
        out_type=jax.ShapeDtypeStruct((NTOK, CP), jnp.float32),
        scratch_types=(
            [pltpu.VMEM((GCH, 128), jnp.int32)]
            + [pltpu.VMEM((128, CP), jnp.float32) for _ in range(NBUF)]
            + [pltpu.SemaphoreType.DMA for _ in range(2 * NBUF)]
        ),
    )


TPB = K_SP * TOPK
TPT = TPB // 16
SCH = TPT // 128
GW = 8
NGP = C // GW
EPT = TPT * GW
ECH = EPT // 128
SPW = HW * GW
EPS = SPW // 16


def _scatter_body(tok_hbm, idxe_hbm, idxc_hbm, consts_hbm, acc_hbm, cnt_hbm,
                  acc_sp, idxe_v, idxc_v, tok_v, zero_v):
    cid = lax.axis_index("c")
    sid = lax.axis_index("s")
    pltpu.sync_copy(idxe_hbm.at[cid, sid], idxe_v)
    pltpu.sync_copy(idxc_hbm.at[cid, sid], idxc_v)
    pltpu.sync_copy(consts_hbm.at[0], zero_v)
    pltpu.sync_copy(consts_hbm.at[1], tok_v.at[pl.ds(0, TPT)])

    for k in range(HW // 16 // TPT):
        pltpu.sync_copy(zero_v,
                        acc_sp.at[pl.ds(sid * (HW // 16) + k * TPT, TPT)])
    plsc.subcore_barrier()
    for j in range(SCH):
        pltpu.sync_copy(tok_v.at[pl.ds(j * 128, 128)],
                        acc_sp.at[idxc_v.at[j]], add=True)
    plsc.subcore_barrier()
    pltpu.sync_copy(acc_sp.at[pl.ds(sid * (HW // 16), HW // 16)],
                    cnt_hbm.at[cid, sid])
    plsc.subcore_barrier()

    for g in range(NGP):
        for k in range(EPS // TPT):
            pltpu.sync_copy(zero_v,
                            acc_sp.at[pl.ds(sid * EPS + k * TPT, TPT)])
        plsc.subcore_barrier()
        pltpu.sync_copy(tok_hbm.at[cid, g, pl.ds(sid * EPT, EPT)], tok_v)

        def chunk(j, _):
            pltpu.sync_copy(tok_v.at[pl.ds(j * 128, 128)],
                            acc_sp.at[idxe_v.at[j]], add=True)
            return 0

        lax.fori_loop(0, ECH, chunk, 0)
        plsc.subcore_barrier()
        pltpu.sync_copy(acc_sp.at[pl.ds(sid * EPS, EPS)],
                        acc_hbm.at[cid, g, sid])
        plsc.subcore_barrier()


@functools.cache
def _scatter_sc():
    return pl.kernel(
        _scatter_body,
        mesh=plsc.VectorSubcoreMesh(core_axis_name="c", subcore_axis_name="s"),
        out_type=[
            jax.ShapeDtypeStruct((B, NGP, 16, EPS), jnp.float32),
            jax.ShapeDtypeStruct((B, 16, HW // 16), jnp.float32),
        ],
        scratch_types=[
            pltpu.VMEM_SHARED((SPW,), jnp.float32),
            pltpu.VMEM((ECH, 128), jnp.int32),
            pltpu.VMEM((SCH, 128), jnp.int32),
            pltpu.VMEM((EPT,), jnp.float32),
            pltpu.VMEM((TPT,), jnp.float32),
        ],
    )


PB = 512
NPIX = B * HW


PPB = HW // PB


def _lnv_body(x_ref, vw_ref, lnw_ref, lnb_ref, xtp_ref, v_ref):
    xc = x_ref[0]
    xcp = jnp.concatenate(
        [xc, jnp.zeros((CP - C, PB), jnp.float32)], axis=0)
    xtpb = jnp.transpose(xcp, (1, 0))
    xt = xtpb[:, :C]
    u = jnp.mean(xt, axis=1, keepdims=True)
    var = jnp.mean((xt - u) ** 2, axis=1, keepdims=True)
    xn = (xt - u) * jax.lax.rsqrt(var + 1e-6)
    xn = xn * lnw_ref[...] + lnb_ref[...]
    dn = (((1,), (1,)), ((), ()))
    v = jax.lax.dot_general(xn, vw_ref[...], dn, preferred_element_type=jnp.float32)
    xtp_ref[...] = xtpb
    v_ref[...] = v


def _lnv(x, v_w, ln_w, ln_b):
    return pl.pallas_call(
        _lnv_body,
        grid=(NPIX // PB,),
        in_specs=[
            pl.BlockSpec((1, C, PB), lambda i: (i // PPB, 0, i % PPB)),
            pl.BlockSpec((C, C), lambda i: (0, 0)),
            pl.BlockSpec((1, C), lambda i: (0, 0)),
            pl.BlockSpec((1, C), lambda i: (0, 0)),
        ],
        out_specs=[
            pl.BlockSpec((PB, CP), lambda i: (i, 0)),
            pl.BlockSpec((PB, C), lambda i: (i, 0)),
        ],
        out_shape=[
            jax.ShapeDtypeStruct((NPIX, CP), jnp.float32),
            jax.ShapeDtypeStruct((NPIX, C), jnp.float32),
        ],
    )(x.reshape(B, C, HW), v_w, ln_w.reshape(1, C), ln_b.reshape(1, C))


def _merge_body(acc_ref, cnt_ref, v_ref, out_ref):
    cnt = cnt_ref[...]
    acc = acc_ref[...]
    mean = acc / jnp.maximum(cnt, 1.0)
    merged = jnp.where(cnt > 1e-5, mean, v_ref[...])
    mp = jnp.concatenate(
        [merged, jnp.zeros((PB, CP - C), jnp.float32)], axis=1)
    mt = jnp.transpose(mp, (1, 0))
    out_ref[0] = mt[:C, :]


def _merge(acc, cnt, v_full):
    return pl.pallas_call(
        _merge_body,
        grid=(NPIX // PB,),
        in_specs=[
            pl.BlockSpec((PB, C), lambda i: (i, 0)),
            pl.BlockSpec((PB, 1), lambda i: (i, 0)),
            pl.BlockSpec((PB, C), lambda i: (i, 0)),
        ],
        out_specs=pl.BlockSpec((1, C, PB), lambda i: (i // PPB, 0, i % PPB)),
        out_shape=jax.ShapeDtypeStruct((B, C, HW), jnp.float32),
    )(acc, cnt.reshape(NPIX, 1), v_full)


SPB = 8
NT = SPB * TOPK


def _attn_body(xg_ref, sims_ref, aw_ref, bqw_ref, bkw_ref, vw_ref,
               lnw_ref, lnb_ref, out_ref):
    xg = xg_ref[:, :C]
    u = jnp.mean(xg, axis=1, keepdims=True)
    var = jnp.mean((xg - u) ** 2, axis=1, keepdims=True)
    xn = (xg - u) * jax.lax.rsqrt(var + 1e-6)
    xn = xn * lnw_ref[...] + lnb_ref[...]
    nn = (((1,), (0,)), ((), ()))
    nt = (((1,), (1,)), ((), ()))
    y = jax.lax.dot_general(xn, aw_ref[...], nn, preferred_element_type=jnp.float32)
    zq = jax.lax.dot_general(xn, bqw_ref[...], nn, preferred_element_type=jnp.float32)
    zk = jax.lax.dot_general(xn, bkw_ref[...], nn, preferred_element_type=jnp.float32)
    v = jax.lax.dot_general(xn, vw_ref[...], nt, preferred_element_type=jnp.float32)
    s_col = sims_ref[...].reshape(NT, 1)
    vw_all = s_col * v
    ones_row = jnp.ones((1, C), dtype=jnp.float32)
    lane = jax.lax.broadcasted_iota(jnp.int32, (1, C), 1)
    qq_cols = []
    zkx = []
    vmask = []
    for h in range(NUM_HEADS):
        cols = slice(h * C, (h + 1) * C)
        qq_cols.append(jnp.sum(zq[:, cols] * xn, axis=1, keepdims=True))
        zkx.append(zk[:, cols] * xn)
        vmask.append(vw_all * (lane // HEAD_DIM == h).astype(jnp.float32))
    for p in range(SPB):
        rows = slice(p * TOPK, (p + 1) * TOPK)
        xr = xn[rows]
        sp_col = s_col[rows]
        parts = []
        for h in range(NUM_HEADS):
            qk = jax.lax.dot_general(y[rows, h * C:(h + 1) * C], xr, nt,
                                     preferred_element_type=jnp.float32)
            kk = jax.lax.dot_general(ones_row, zkx[h][rows], nt,
                                     preferred_element_type=jnp.float32)
            d2 = qq_cols[h][rows] + kk - 2.0 * qk
            dist = jnp.sqrt(jnp.maximum(d2, 1e-12))
            a = -SCALE * dist
            m = jnp.max(a, axis=1, keepdims=True)
            e = jnp.exp(a - m)
            parts.append(e / jnp.sum(e, axis=1, keepdims=True))
        attn_cat = jnp.concatenate(parts, axis=1)
        vbd = jnp.concatenate([vmask[h][rows] for h in range(NUM_HEADS)],
                              axis=0)
        oh = jax.lax.dot_general(attn_cat, vbd, nn,
                                 preferred_element_type=jnp.float32)
        out_ref[rows, :] = sp_col * oh


def _attention(xg2, simsT, q_w, k_w, v_w, ln_w, ln_b):
    heads_q = q_w.reshape(NUM_HEADS, HEAD_DIM, C)
    heads_k = k_w.reshape(NUM_HEADS, HEAD_DIM, C)
    a_cat = jnp.concatenate(
        [heads_q[h].T @ heads_k[h] for h in range(NUM_HEADS)], axis=1)
    bq_cat = jnp.concatenate(
        [heads_q[h].T @ heads_q[h] for h in range(NUM_HEADS)], axis=1)
    bk_cat = jnp.concatenate(
        [heads_k[h].T @ heads_k[h] for h in range(NUM_HEADS)], axis=1)
    return pl.pallas_call(
        _attn_body,
        grid=(NSP // SPB,),
        in_specs=[
            pl.BlockSpec((NT, CP), lambda i: (i, 0)),
            pl.BlockSpec((SPB, TOPK, 1), lambda i: (i, 0, 0)),
            pl.BlockSpec((C, NUM_HEADS * C), lambda i: (0, 0)),
            pl.BlockSpec((C, NUM_HEADS * C), lambda i: (0, 0)),
            pl.BlockSpec((C, NUM_HEADS * C), lambda i: (0, 0)),
            pl.BlockSpec((C, C), lambda i: (0, 0)),
            pl.BlockSpec((1, C), lambda i: (0, 0)),
            pl.BlockSpec((1, C), lambda i: (0, 0)),
        ],
        out_specs=pl.BlockSpec((NT, C), lambda i: (i, 0)),
        out_shape=jax.ShapeDtypeStruct((NSP * TOPK, C), jnp.float32),
    )(xg2, simsT, a_cat, bq_cat, bk_cat, v_w, ln_w.reshape(1, C),
      ln_b.reshape(1, C))


def kernel(x, sims, mask, ln_w, ln_b, q_w, k_w, v_w, indices, labels, num_spixels):
    idx = indices.reshape(B, K_SP * TOPK)
    idx_g = (idx + jnp.arange(B, dtype=jnp.int32)[:, None] * HW)
    xtp, v_full = _lnv(x, v_w, ln_w, ln_b)
    xg = _gather_sc()(xtp, idx_g.reshape(NW, GCH, 128))

    out_tok = _attention(
        xg,
        sims.reshape(NSP, TOPK, 1),
        q_w, k_w, v_w, ln_w.reshape(1, C), ln_b.reshape(1, C))
    out_cg = (out_tok.reshape(B, TPB, NGP, GW)
              .transpose(0, 2, 1, 3).reshape(B, NGP, TPB * GW))
    idx_e = (idx[..., None] * GW
             + jnp.arange(GW, dtype=jnp.int32)).reshape(B, 16, ECH, 128)
    idx_c = idx.reshape(B, 16, SCH, 128)
    consts = jnp.stack([jnp.zeros((TPT,), jnp.float32),
                        jnp.ones((TPT,), jnp.float32)])
    acc_f, cnt_f = _scatter_sc()(out_cg, idx_e, idx_c, consts)
    acc = (acc_f.reshape(B, NGP, HW, GW)
           .transpose(0, 2, 1, 3).reshape(NPIX, C))
    cnt = cnt_f.reshape(NPIX, 1)
    mean = acc / jnp.maximum(cnt, 1.0)
    merged = jnp.where(cnt > 1e-5, mean, v_full)
    return (merged.reshape(B, HW, C).transpose(0, 2, 1)
            .reshape(B, C, H, W))


# speedup vs baseline: 1.2572x; 1.0526x over previous
import functools

import jax
import jax.numpy as jnp
from jax import lax
from jax.experimental import pallas as pl
from jax.experimental.pallas import tpu as pltpu
from jax.experimental.pallas import tpu_sc as plsc

B, C, H, W = 2, 96, 384, 384
QK_DIM = 96
NUM_HEADS = 3
K_SP = 576
TOPK = 64
HEAD_DIM = QK_DIM // NUM_HEADS
SCALE = HEAD_DIM ** (-0.5)
HW = H * W
NSP = B * K_SP


NTOK = B * K_SP * TOPK
NW = 32
TPW = NTOK // NW
GCH = 18
assert GCH * 128 == TPW


CP = 128
NBUF = 4


def _gather_body(xt_hbm, idxg_hbm, out_hbm, idx_v, *bufsems):
    cid = lax.axis_index("c")
    sid = lax.axis_index("s")
    wid = sid * 2 + cid
    pltpu.sync_copy(idxg_hbm.at[wid], idx_v)
    bufs = bufsems[:NBUF]
    gsems = bufsems[NBUF:2 * NBUF]
    wsems = bufsems[2 * NBUF:]

    def out_slice(j):
        return out_hbm.at[pl.ds(wid * TPW + j * 128, 128)]

    for j in range(NBUF):
        pltpu.async_copy(xt_hbm.at[idx_v.at[j]], bufs[j], gsems[j])
    for j in range(GCH):
        b = j % NBUF
        pltpu.make_async_copy(xt_hbm.at[idx_v.at[j]], bufs[b], gsems[b]).wait()
        pltpu.async_copy(bufs[b], out_slice(j), wsems[b])
        if j + NBUF < GCH:
            pltpu.make_async_copy(bufs[b], out_slice(j), wsems[b]).wait()
            pltpu.async_copy(xt_hbm.at[idx_v.at[j + NBUF]], bufs[b], gsems[b])
    for j in range(GCH - NBUF, GCH):
        b = j % NBUF
        pltpu.make_async_copy(bufs[b], out_slice(j), wsems[b]).wait()


@functools.cache
def _gather_sc():
    return pl.kernel(
        _gather_body,
        mesh=plsc.VectorSubcoreMesh(core_axis_name="c", subcore_axis_name="s"),
        out_type=jax.ShapeDtypeStruct((NTOK, CP), jnp.float32),
        scratch_types=(
            [pltpu.VMEM((GCH, 128), jnp.int32)]
            + [pltpu.VMEM((128, CP), jnp.float32) for _ in range(NBUF)]
            + [pltpu.SemaphoreType.DMA for _ in range(2 * NBUF)]
        ),
    )


TPB = K_SP * TOPK
TPT = TPB // 16
SCH = TPT // 128
GW = 8
NGP = C // GW
EPT = TPT * GW
ECH = EPT // 128
SPW = HW * GW
EPS = SPW // 16


def _scatter_body(tok_hbm, idxe_hbm, idxc_hbm, consts_hbm, acc_hbm, cnt_hbm,
                  acc_sp, idxe_v, idxc_v, tok_v, zero_v):
    cid = lax.axis_index("c")
    sid = lax.axis_index("s")
    pltpu.sync_copy(idxe_hbm.at[cid, sid], idxe_v)
    pltpu.sync_copy(idxc_hbm.at[cid, sid], idxc_v)
    pltpu.sync_copy(consts_hbm.at[0], zero_v)
    pltpu.sync_copy(consts_hbm.at[1], tok_v.at[pl.ds(0, TPT)])

    for k in range(HW // 16 // TPT):
        pltpu.sync_copy(zero_v,
                        acc_sp.at[pl.ds(sid * (HW // 16) + k * TPT, TPT)])
    plsc.subcore_barrier()
    for j in range(SCH):
        pltpu.sync_copy(tok_v.at[pl.ds(j * 128, 128)],
                        acc_sp.at[idxc_v.at[j]], add=True)
    plsc.subcore_barrier()
    pltpu.sync_copy(acc_sp.at[pl.ds(sid * (HW // 16), HW // 16)],
                    cnt_hbm.at[cid, sid])
    plsc.subcore_barrier()

    for g in range(NGP):
        for k in range(EPS // TPT):
            pltpu.sync_copy(zero_v,
                            acc_sp.at[pl.ds(sid * EPS + k * TPT, TPT)])
        plsc.subcore_barrier()
        pltpu.sync_copy(tok_hbm.at[cid, g, pl.ds(sid * EPT, EPT)], tok_v)

        def chunk(j, _):
            pltpu.sync_copy(tok_v.at[pl.ds(j * 128, 128)],
                            acc_sp.at[idxe_v.at[j]], add=True)
            return 0

        lax.fori_loop(0, ECH, chunk, 0)
        plsc.subcore_barrier()
        pltpu.sync_copy(acc_sp.at[pl.ds(sid * EPS, EPS)],
                        acc_hbm.at[cid, g, sid])
        plsc.subcore_barrier()


@functools.cache
def _scatter_sc():
    return pl.kernel(
        _scatter_body,
        mesh=plsc.VectorSubcoreMesh(core_axis_name="c", subcore_axis_name="s"),
        out_type=[
            jax.ShapeDtypeStruct((B, NGP, 16, EPS), jnp.float32),
            jax.ShapeDtypeStruct((B, 16, HW // 16), jnp.float32),
        ],
        scratch_types=[
            pltpu.VMEM_SHARED((SPW,), jnp.float32),
            pltpu.VMEM((ECH, 128), jnp.int32),
            pltpu.VMEM((SCH, 128), jnp.int32),
            pltpu.VMEM((EPT,), jnp.float32),
            pltpu.VMEM((TPT,), jnp.float32),
        ],
    )


PB = 2048
NPIX = B * HW


PPB = HW // PB


def _lnv_body(x_ref, vw_ref, lnw_ref, lnb_ref, xtp_ref, v_ref):
    xc = x_ref[0]
    xcp = jnp.concatenate(
        [xc, jnp.zeros((CP - C, PB), jnp.float32)], axis=0)
    xtpb = jnp.transpose(xcp, (1, 0))
    xt = xtpb[:, :C]
    u = jnp.mean(xt, axis=1, keepdims=True)
    var = jnp.mean((xt - u) ** 2, axis=1, keepdims=True)
    xn = (xt - u) * jax.lax.rsqrt(var + 1e-6)
    xn = xn * lnw_ref[...] + lnb_ref[...]
    dn = (((1,), (1,)), ((), ()))
    v = jax.lax.dot_general(xn, vw_ref[...], dn, preferred_element_type=jnp.float32)
    xtp_ref[...] = xtpb
    v_ref[...] = v


def _lnv(x, v_w, ln_w, ln_b):
    return pl.pallas_call(
        _lnv_body,
        grid=(NPIX // PB,),
        in_specs=[
            pl.BlockSpec((1, C, PB), lambda i: (i // PPB, 0, i % PPB)),
            pl.BlockSpec((C, C), lambda i: (0, 0)),
            pl.BlockSpec((1, C), lambda i: (0, 0)),
            pl.BlockSpec((1, C), lambda i: (0, 0)),
        ],
        out_specs=[
            pl.BlockSpec((PB, CP), lambda i: (i, 0)),
            pl.BlockSpec((PB, C), lambda i: (i, 0)),
        ],
        out_shape=[
            jax.ShapeDtypeStruct((NPIX, CP), jnp.float32),
            jax.ShapeDtypeStruct((NPIX, C), jnp.float32),
        ],
    )(x.reshape(B, C, HW), v_w, ln_w.reshape(1, C), ln_b.reshape(1, C))


def _merge_body(acc_ref, cnt_ref, v_ref, out_ref):
    cnt = cnt_ref[...]
    acc = acc_ref[...]
    mean = acc / jnp.maximum(cnt, 1.0)
    merged = jnp.where(cnt > 1e-5, mean, v_ref[...])
    mp = jnp.concatenate(
        [merged, jnp.zeros((PB, CP - C), jnp.float32)], axis=1)
    mt = jnp.transpose(mp, (1, 0))
    out_ref[0] = mt[:C, :]


def _merge(acc, cnt, v_full):
    return pl.pallas_call(
        _merge_body,
        grid=(NPIX // PB,),
        in_specs=[
            pl.BlockSpec((PB, C), lambda i: (i, 0)),
            pl.BlockSpec((PB, 1), lambda i: (i, 0)),
            pl.BlockSpec((PB, C), lambda i: (i, 0)),
        ],
        out_specs=pl.BlockSpec((1, C, PB), lambda i: (i // PPB, 0, i % PPB)),
        out_shape=jax.ShapeDtypeStruct((B, C, HW), jnp.float32),
    )(acc, cnt.reshape(NPIX, 1), v_full)


SPB = 8
NT = SPB * TOPK


def _attn_body(xg_ref, sims_ref, aw_ref, bqw_ref, bkw_ref, vw_ref,
               lnw_ref, lnb_ref, out_ref):
    xg = xg_ref[:, :C]
    u = jnp.mean(xg, axis=1, keepdims=True)
    var = jnp.mean((xg - u) ** 2, axis=1, keepdims=True)
    xn = (xg - u) * jax.lax.rsqrt(var + 1e-6)
    xn = xn * lnw_ref[...] + lnb_ref[...]
    nn = (((1,), (0,)), ((), ()))
    nt = (((1,), (1,)), ((), ()))
    y = jax.lax.dot_general(xn, aw_ref[...], nn, preferred_element_type=jnp.float32)
    zq = jax.lax.dot_general(xn, bqw_ref[...], nn, preferred_element_type=jnp.float32)
    zk = jax.lax.dot_general(xn, bkw_ref[...], nn, preferred_element_type=jnp.float32)
    v = jax.lax.dot_general(xn, vw_ref[...], nt, preferred_element_type=jnp.float32)
    s_col = sims_ref[...].reshape(NT, 1)
    vw_all = s_col * v
    ones_row = jnp.ones((1, C), dtype=jnp.float32)
    lane = jax.lax.broadcasted_iota(jnp.int32, (1, C), 1)
    qq_cols = []
    zkx = []
    vmask = []
    for h in range(NUM_HEADS):
        cols = slice(h * C, (h + 1) * C)
        qq_cols.append(jnp.sum(zq[:, cols] * xn, axis=1, keepdims=True))
        zkx.append(zk[:, cols] * xn)
        vmask.append(vw_all * (lane // HEAD_DIM == h).astype(jnp.float32))
    for p in range(SPB):
        rows = slice(p * TOPK, (p + 1) * TOPK)
        xr = xn[rows]
        sp_col = s_col[rows]
        parts = []
        for h in range(NUM_HEADS):
            qk = jax.lax.dot_general(y[rows, h * C:(h + 1) * C], xr, nt,
                                     preferred_element_type=jnp.float32)
            kk = jax.lax.dot_general(ones_row, zkx[h][rows], nt,
                                     preferred_element_type=jnp.float32)
            d2 = qq_cols[h][rows] + kk - 2.0 * qk
            dist = jnp.sqrt(jnp.maximum(d2, 1e-12))
            a = -SCALE * dist
            m = jnp.max(a, axis=1, keepdims=True)
            e = jnp.exp(a - m)
            parts.append(e / jnp.sum(e, axis=1, keepdims=True))
        attn_cat = jnp.concatenate(parts, axis=1)
        vbd = jnp.concatenate([vmask[h][rows] for h in range(NUM_HEADS)],
                              axis=0)
        oh = jax.lax.dot_general(attn_cat, vbd, nn,
                                 preferred_element_type=jnp.float32)
        out_ref[rows, :] = sp_col * oh


def _attention(xg2, simsT, q_w, k_w, v_w, ln_w, ln_b):
    heads_q = q_w.reshape(NUM_HEADS, HEAD_DIM, C)
    heads_k = k_w.reshape(NUM_HEADS, HEAD_DIM, C)
    a_cat = jnp.concatenate(
        [heads_q[h].T @ heads_k[h] for h in range(NUM_HEADS)], axis=1)
    bq_cat = jnp.concatenate(
        [heads_q[h].T @ heads_q[h] for h in range(NUM_HEADS)], axis=1)
    bk_cat = jnp.concatenate(
        [heads_k[h].T @ heads_k[h] for h in range(NUM_HEADS)], axis=1)
    return pl.pallas_call(
        _attn_body,
        grid=(NSP // SPB,),
        in_specs=[
            pl.BlockSpec((NT, CP), lambda i: (i, 0)),
            pl.BlockSpec((SPB, TOPK, 1), lambda i: (i, 0, 0)),
            pl.BlockSpec((C, NUM_HEADS * C), lambda i: (0, 0)),
            pl.BlockSpec((C, NUM_HEADS * C), lambda i: (0, 0)),
            pl.BlockSpec((C, NUM_HEADS * C), lambda i: (0, 0)),
            pl.BlockSpec((C, C), lambda i: (0, 0)),
            pl.BlockSpec((1, C), lambda i: (0, 0)),
            pl.BlockSpec((1, C), lambda i: (0, 0)),
        ],
        out_specs=pl.BlockSpec((NT, C), lambda i: (i, 0)),
        out_shape=jax.ShapeDtypeStruct((NSP * TOPK, C), jnp.float32),
    )(xg2, simsT, a_cat, bq_cat, bk_cat, v_w, ln_w.reshape(1, C),
      ln_b.reshape(1, C))


def kernel(x, sims, mask, ln_w, ln_b, q_w, k_w, v_w, indices, labels, num_spixels):
    idx = indices.reshape(B, K_SP * TOPK)
    idx_g = (idx + jnp.arange(B, dtype=jnp.int32)[:, None] * HW)
    xtp, v_full = _lnv(x, v_w, ln_w, ln_b)
    xg = _gather_sc()(xtp, idx_g.reshape(NW, GCH, 128))

    out_tok = _attention(
        xg,
        sims.reshape(NSP, TOPK, 1),
        q_w, k_w, v_w, ln_w.reshape(1, C), ln_b.reshape(1, C))
    out_cg = (out_tok.reshape(B, TPB, NGP, GW)
              .transpose(0, 2, 1, 3).reshape(B, NGP, TPB * GW))
    idx_e = (idx[..., None] * GW
             + jnp.arange(GW, dtype=jnp.int32)).reshape(B, 16, ECH, 128)
    idx_c = idx.reshape(B, 16, SCH, 128)
    consts = jnp.stack([jnp.zeros((TPT,), jnp.float32),
                        jnp.ones((TPT,), jnp.float32)])
    acc_f, cnt_f = _scatter_sc()(out_cg, idx_e, idx_c, consts)
    acc = (acc_f.reshape(B, NGP, HW, GW)
           .transpose(0, 2, 1, 3).reshape(NPIX, C))
    cnt = cnt_f.reshape(NPIX, 1)
    mean = acc / jnp.maximum(cnt, 1.0)
    merged = jnp.where(cnt > 1e-5, mean, v_full)
    return (merged.reshape(B, HW, C).transpose(0, 2, 1)
            .reshape(B, C, H, W))


# speedup vs baseline: 1.2579x; 1.0006x over previous
import functools

import jax
import jax.numpy as jnp
from jax import lax
from jax.experimental import pallas as pl
from jax.experimental.pallas import tpu as pltpu
from jax.experimental.pallas import tpu_sc as plsc

B, C, H, W = 2, 96, 384, 384
QK_DIM = 96
NUM_HEADS = 3
K_SP = 576
TOPK = 64
HEAD_DIM = QK_DIM // NUM_HEADS
SCALE = HEAD_DIM ** (-0.5)
HW = H * W
NSP = B * K_SP


NTOK = B * K_SP * TOPK
NW = 32
TPW = NTOK // NW
GCH = 18
assert GCH * 128 == TPW


CP = 128
NBUF = 4


def _gather_body(xt_hbm, idxg_hbm, out_hbm, idx_v, *bufsems):
    cid = lax.axis_index("c")
    sid = lax.axis_index("s")
    wid = sid * 2 + cid
    pltpu.sync_copy(idxg_hbm.at[wid], idx_v)
    bufs = bufsems[:NBUF]
    gsems = bufsems[NBUF:2 * NBUF]
    wsems = bufsems[2 * NBUF:]

    def out_slice(j):
        return out_hbm.at[pl.ds(wid * TPW + j * 128, 128)]

    for j in range(NBUF):
        pltpu.async_copy(xt_hbm.at[idx_v.at[j]], bufs[j], gsems[j])
    for j in range(GCH):
        b = j % NBUF
        pltpu.make_async_copy(xt_hbm.at[idx_v.at[j]], bufs[b], gsems[b]).wait()
        pltpu.async_copy(bufs[b], out_slice(j), wsems[b])
        if j + NBUF < GCH:
            pltpu.make_async_copy(bufs[b], out_slice(j), wsems[b]).wait()
            pltpu.async_copy(xt_hbm.at[idx_v.at[j + NBUF]], bufs[b], gsems[b])
    for j in range(GCH - NBUF, GCH):
        b = j % NBUF
        pltpu.make_async_copy(bufs[b], out_slice(j), wsems[b]).wait()


@functools.cache
def _gather_sc():
    return pl.kernel(
        _gather_body,
        mesh=plsc.VectorSubcoreMesh(core_axis_name="c", subcore_axis_name="s"),
        out_type=jax.ShapeDtypeStruct((NTOK, CP), jnp.float32),
        scratch_types=(
            [pltpu.VMEM((GCH, 128), jnp.int32)]
            + [pltpu.VMEM((128, CP), jnp.float32) for _ in range(NBUF)]
            + [pltpu.SemaphoreType.DMA for _ in range(2 * NBUF)]
        ),
    )


TPB = K_SP * TOPK
TPT = TPB // 16
SCH = TPT // 128
GW = 8
NGP = C // GW
EPT = TPT * GW
ECH = EPT // 128
SPW = HW * GW
EPS = SPW // 16


def _scatter_body(tok_hbm, idxe_hbm, idxc_hbm, consts_hbm, acc_hbm, cnt_hbm,
                  acc_sp, idxe_v, idxc_v, tok_v, zero_v):
    cid = lax.axis_index("c")
    sid = lax.axis_index("s")
    pltpu.sync_copy(idxe_hbm.at[cid, sid], idxe_v)
    pltpu.sync_copy(idxc_hbm.at[cid, sid], idxc_v)
    pltpu.sync_copy(consts_hbm.at[0], zero_v)
    pltpu.sync_copy(consts_hbm.at[1], tok_v.at[pl.ds(0, TPT)])

    for k in range(HW // 16 // TPT):
        pltpu.sync_copy(zero_v,
                        acc_sp.at[pl.ds(sid * (HW // 16) + k * TPT, TPT)])
    plsc.subcore_barrier()
    for j in range(SCH):
        pltpu.sync_copy(tok_v.at[pl.ds(j * 128, 128)],
                        acc_sp.at[idxc_v.at[j]], add=True)
    plsc.subcore_barrier()
    pltpu.sync_copy(acc_sp.at[pl.ds(sid * (HW // 16), HW // 16)],
                    cnt_hbm.at[cid, sid])
    plsc.subcore_barrier()

    for g in range(NGP):
        for k in range(EPS // TPT):
            pltpu.sync_copy(zero_v,
                            acc_sp.at[pl.ds(sid * EPS + k * TPT, TPT)])
        plsc.subcore_barrier()
        pltpu.sync_copy(tok_hbm.at[cid, g, pl.ds(sid * EPT, EPT)], tok_v)

        def chunk(j, _):
            pltpu.sync_copy(tok_v.at[pl.ds(j * 128, 128)],
                            acc_sp.at[idxe_v.at[j]], add=True)
            return 0

        lax.fori_loop(0, ECH, chunk, 0)
        plsc.subcore_barrier()
        pltpu.sync_copy(acc_sp.at[pl.ds(sid * EPS, EPS)],
                        acc_hbm.at[cid, g, sid])
        plsc.subcore_barrier()


@functools.cache
def _scatter_sc():
    return pl.kernel(
        _scatter_body,
        mesh=plsc.VectorSubcoreMesh(core_axis_name="c", subcore_axis_name="s"),
        out_type=[
            jax.ShapeDtypeStruct((B, NGP, 16, EPS), jnp.float32),
            jax.ShapeDtypeStruct((B, 16, HW // 16), jnp.float32),
        ],
        scratch_types=[
            pltpu.VMEM_SHARED((SPW,), jnp.float32),
            pltpu.VMEM((ECH, 128), jnp.int32),
            pltpu.VMEM((SCH, 128), jnp.int32),
            pltpu.VMEM((EPT,), jnp.float32),
            pltpu.VMEM((TPT,), jnp.float32),
        ],
    )


PB = 2048
NPIX = B * HW


PPB = HW // PB


def _lnv_body(x_ref, vw_ref, lnw_ref, lnb_ref, xtp_ref, v_ref):
    xc = x_ref[0]
    xcp = jnp.concatenate(
        [xc, jnp.zeros((CP - C, PB), jnp.float32)], axis=0)
    xtpb = jnp.transpose(xcp, (1, 0))
    xt = xtpb[:, :C]
    u = jnp.mean(xt, axis=1, keepdims=True)
    var = jnp.mean((xt - u) ** 2, axis=1, keepdims=True)
    xn = (xt - u) * jax.lax.rsqrt(var + 1e-6)
    xn = xn * lnw_ref[...] + lnb_ref[...]
    dn = (((1,), (1,)), ((), ()))
    v = jax.lax.dot_general(xn, vw_ref[...], dn, preferred_element_type=jnp.float32)
    xtp_ref[...] = xtpb
    v_ref[...] = v


def _lnv(x, v_w, ln_w, ln_b):
    return pl.pallas_call(
        _lnv_body,
        grid=(NPIX // PB,),
        in_specs=[
            pl.BlockSpec((1, C, PB), lambda i: (i // PPB, 0, i % PPB)),
            pl.BlockSpec((C, C), lambda i: (0, 0)),
            pl.BlockSpec((1, C), lambda i: (0, 0)),
            pl.BlockSpec((1, C), lambda i: (0, 0)),
        ],
        out_specs=[
            pl.BlockSpec((PB, CP), lambda i: (i, 0)),
            pl.BlockSpec((PB, C), lambda i: (i, 0)),
        ],
        out_shape=[
            jax.ShapeDtypeStruct((NPIX, CP), jnp.float32),
            jax.ShapeDtypeStruct((NPIX, C), jnp.float32),
        ],
    )(x.reshape(B, C, HW), v_w, ln_w.reshape(1, C), ln_b.reshape(1, C))


SPB = 8
NT = SPB * TOPK


def _attn_body(xg_ref, sims_ref, aw_ref, bqw_ref, bkw_ref, vw_ref,
               lnw_ref, lnb_ref, out_ref):
    xg = xg_ref[:, :C]
    u = jnp.mean(xg, axis=1, keepdims=True)
    var = jnp.mean((xg - u) ** 2, axis=1, keepdims=True)
    xn = (xg - u) * jax.lax.rsqrt(var + 1e-6)
    xn = xn * lnw_ref[...] + lnb_ref[...]
    nn = (((1,), (0,)), ((), ()))
    nt = (((1,), (1,)), ((), ()))
    y = jax.lax.dot_general(xn, aw_ref[...], nn, preferred_element_type=jnp.float32)
    zq = jax.lax.dot_general(xn, bqw_ref[...], nn, preferred_element_type=jnp.float32)
    zk = jax.lax.dot_general(xn, bkw_ref[...], nn, preferred_element_type=jnp.float32)
    v = jax.lax.dot_general(xn, vw_ref[...], nt, preferred_element_type=jnp.float32)
    s_col = sims_ref[...].reshape(NT, 1)
    vw_all = s_col * v
    ones_row = jnp.ones((1, C), dtype=jnp.float32)
    lane = jax.lax.broadcasted_iota(jnp.int32, (1, C), 1)
    qq_cols = []
    zkx = []
    vmask = []
    for h in range(NUM_HEADS):
        cols = slice(h * C, (h + 1) * C)
        qq_cols.append(jnp.sum(zq[:, cols] * xn, axis=1, keepdims=True))
        zkx.append(zk[:, cols] * xn)
        vmask.append(vw_all * (lane // HEAD_DIM == h).astype(jnp.float32))
    for p in range(SPB):
        rows = slice(p * TOPK, (p + 1) * TOPK)
        xr = xn[rows]
        sp_col = s_col[rows]
        parts = []
        for h in range(NUM_HEADS):
            qk = jax.lax.dot_general(y[rows, h * C:(h + 1) * C], xr, nt,
                                     preferred_element_type=jnp.float32)
            kk = jax.lax.dot_general(ones_row, zkx[h][rows], nt,
                                     preferred_element_type=jnp.float32)
            d2 = qq_cols[h][rows] + kk - 2.0 * qk
            dist = jnp.sqrt(jnp.maximum(d2, 1e-12))
            a = -SCALE * dist
            m = jnp.max(a, axis=1, keepdims=True)
            e = jnp.exp(a - m)
            parts.append(e / jnp.sum(e, axis=1, keepdims=True))
        attn_cat = jnp.concatenate(parts, axis=1)
        vbd = jnp.concatenate([vmask[h][rows] for h in range(NUM_HEADS)],
                              axis=0)
        oh = jax.lax.dot_general(attn_cat, vbd, nn,
                                 preferred_element_type=jnp.float32)
        out_ref[rows, :] = sp_col * oh


def _attention(xg2, simsT, q_w, k_w, v_w, ln_w, ln_b):
    heads_q = q_w.reshape(NUM_HEADS, HEAD_DIM, C)
    heads_k = k_w.reshape(NUM_HEADS, HEAD_DIM, C)
    a_cat = jnp.concatenate(
        [heads_q[h].T @ heads_k[h] for h in range(NUM_HEADS)], axis=1)
    bq_cat = jnp.concatenate(
        [heads_q[h].T @ heads_q[h] for h in range(NUM_HEADS)], axis=1)
    bk_cat = jnp.concatenate(
        [heads_k[h].T @ heads_k[h] for h in range(NUM_HEADS)], axis=1)
    return pl.pallas_call(
        _attn_body,
        grid=(NSP // SPB,),
        in_specs=[
            pl.BlockSpec((NT, CP), lambda i: (i, 0)),
            pl.BlockSpec((SPB, TOPK, 1), lambda i: (i, 0, 0)),
            pl.BlockSpec((C, NUM_HEADS * C), lambda i: (0, 0)),
            pl.BlockSpec((C, NUM_HEADS * C), lambda i: (0, 0)),
            pl.BlockSpec((C, NUM_HEADS * C), lambda i: (0, 0)),
            pl.BlockSpec((C, C), lambda i: (0, 0)),
            pl.BlockSpec((1, C), lambda i: (0, 0)),
            pl.BlockSpec((1, C), lambda i: (0, 0)),
        ],
        out_specs=pl.BlockSpec((NT, C), lambda i: (i, 0)),
        out_shape=jax.ShapeDtypeStruct((NSP * TOPK, C), jnp.float32),
    )(xg2, simsT, a_cat, bq_cat, bk_cat, v_w, ln_w.reshape(1, C),
      ln_b.reshape(1, C))


def kernel(x, sims, mask, ln_w, ln_b, q_w, k_w, v_w, indices, labels, num_spixels):
    idx = indices.reshape(B, K_SP * TOPK)
    idx_g = (idx + jnp.arange(B, dtype=jnp.int32)[:, None] * HW)
    xtp, v_full = _lnv(x, v_w, ln_w, ln_b)
    xg = _gather_sc()(xtp, idx_g.reshape(NW, GCH, 128))

    out_tok = _attention(
        xg,
        sims.reshape(NSP, TOPK, 1),
        q_w, k_w, v_w, ln_w.reshape(1, C), ln_b.reshape(1, C))
    out_cg = (out_tok.reshape(B, TPB, NGP, GW)
              .transpose(0, 2, 1, 3).reshape(B, NGP, TPB * GW))
    idx_e = (idx[..., None] * GW
             + jnp.arange(GW, dtype=jnp.int32)).reshape(B, 16, ECH, 128)
    idx_c = idx.reshape(B, 16, SCH, 128)
    consts = jnp.stack([jnp.zeros((TPT,), jnp.float32),
                        jnp.ones((TPT,), jnp.float32)])
    acc_f, cnt_f = _scatter_sc()(out_cg, idx_e, idx_c, consts)
    acc = (acc_f.reshape(B, NGP, HW, GW)
           .transpose(0, 2, 1, 3).reshape(NPIX, C))
    cnt = cnt_f.reshape(NPIX, 1)
    mean = acc / jnp.maximum(cnt, 1.0)
    merged = jnp.where(cnt > 1e-5, mean, v_full)
    return (merged.reshape(B, HW, C).transpose(0, 2, 1)
            .reshape(B, C, H, W))
